# Initial kernel scaffold; baseline (speedup 1.0000x reference)
import jax
import jax.numpy as jnp
from jax.experimental import pallas as pl


def kernel(x, params):
    raise NotImplementedError("write your pallas kernel here")


# speedup vs baseline: 3.7040x; 3.7040x over previous
import functools
import math

import jax
import jax.numpy as jnp
from jax import lax
from jax.experimental import pallas as pl
from jax.experimental.pallas import tpu as pltpu

_BIGF = 3.0e38
_NEG = -1.0e30
_PAD = 1.0e9


def _relu(v):
    return jnp.maximum(v, 0.0)


def _dot(a, b):
    return jnp.dot(a, b, preferred_element_type=jnp.float32)


def _fps_kernel(px_ref, py_ref, ox_ref, oy_ref, *, n_valid, num_sel):
    R = px_ref.shape[0]
    OR = ox_ref.shape[0]
    iota = (lax.broadcasted_iota(jnp.int32, (R, 128), 0) * 128
            + lax.broadcasted_iota(jnp.int32, (R, 128), 1))
    oiota = (lax.broadcasted_iota(jnp.int32, (OR, 128), 0) * 128
             + lax.broadcasted_iota(jnp.int32, (OR, 128), 1))
    px = px_ref[...]
    py = py_ref[...]
    x0 = px[0, 0]
    y0 = py[0, 0]
    d0 = (px - x0) ** 2 + (py - y0) ** 2
    dists = jnp.where(iota < n_valid, d0, -1.0)
    selx = jnp.where(oiota == 0, x0, 0.0)
    sely = jnp.where(oiota == 0, y0, 0.0)

    def body(i, carry):
        dists, selx, sely = carry
        m = jnp.max(dists)
        nxt = jnp.min(jnp.where(dists == m, iota, jnp.int32(2 ** 30)))
        gx = jnp.sum(jnp.where(iota == nxt, px, 0.0))
        gy = jnp.sum(jnp.where(iota == nxt, py, 0.0))
        d = (px - gx) ** 2 + (py - gy) ** 2
        dists = jnp.minimum(dists, d)
        selx = jnp.where(oiota == i, gx, selx)
        sely = jnp.where(oiota == i, gy, sely)
        return dists, selx, sely

    dists, selx, sely = lax.fori_loop(1, num_sel, body, (dists, selx, sely))
    ox_ref[...] = selx
    oy_ref[...] = sely


def _fps_call(px, py, n_valid, num_sel, out_rows):
    kfn = functools.partial(_fps_kernel, n_valid=n_valid, num_sel=num_sel)
    out_sds = jax.ShapeDtypeStruct((out_rows, 128), jnp.float32)
    return pl.pallas_call(
        kfn,
        out_shape=(out_sds, out_sds),
    )(px, py)


def _topk_kernel(qx_ref, qy_ref, sx_ref, sy_ref, idx_ref, dsel_ref, d2_ref,
                 *, k):
    S = sx_ref.shape[1]
    qx = qx_ref[...]
    qy = qy_ref[...]
    sx = sx_ref[...]
    sy = sy_ref[...]
    d2_ref[...] = (qx - sx) ** 2 + (qy - sy) ** 2
    siota = lax.broadcasted_iota(jnp.int32, (1, S), 1)
    for r in range(k):
        d2 = d2_ref[...]
        best = jnp.min(d2, axis=1, keepdims=True)
        bidx = jnp.min(jnp.where(d2 == best, siota, jnp.int32(2 ** 30)),
                       axis=1, keepdims=True)
        idx_ref[:, r:r + 1] = bidx
        dsel_ref[:, r:r + 1] = best
        d2_ref[...] = jnp.where(siota == bidx, _BIGF, d2)


def _topk_call(qx, qy, sx, sy, k):
    NQ = qx.shape[0]
    S = sx.shape[1]
    nb = NQ // 128
    kfn = functools.partial(_topk_kernel, k=k)
    qspec = pl.BlockSpec((128, 1), lambda i: (i, 0))
    sspec = pl.BlockSpec((1, S), lambda i: (0, 0))
    return pl.pallas_call(
        kfn,
        grid=(nb,),
        in_specs=[qspec, qspec, sspec, sspec],
        out_specs=(pl.BlockSpec((128, k), lambda i: (i, 0)),
                   pl.BlockSpec((128, k), lambda i: (i, 0))),
        out_shape=(jax.ShapeDtypeStruct((NQ, k), jnp.int32),
                   jax.ShapeDtypeStruct((NQ, k), jnp.float32)),
        scratch_shapes=[pltpu.VMEM((128, S), jnp.float32)],
    )(qx, qy, sx, sy)


def _conv_kernel(xin_ref, dsel_ref, w1_ref, b1_ref, w2_ref, b2_ref,
                 g1_ref, c1_ref, g2_ref, c2_ref, o_ref, *, r2, kn):
    h = _relu(_dot(xin_ref[...], w1_ref[...]) + b1_ref[...])
    h = _relu(_dot(h, w2_ref[...]) + b2_ref[...])
    C2 = h.shape[1]
    h3 = h.reshape(128, kn, C2)
    mask = dsel_ref[...] <= r2
    hm = jnp.full((128, C2), _NEG, jnp.float32)
    for j in range(kn):
        hm = jnp.maximum(hm, jnp.where(mask[:, j:j + 1], h3[:, j, :], _NEG))
    g = _relu(_dot(hm, g1_ref[...]) + c1_ref[...])
    o_ref[...] = _relu(_dot(g, g2_ref[...]) + c2_ref[...])


def _conv_call(xin, dsel, loc, glob, r2):
    NQ, kn = dsel.shape
    nb = NQ // 128
    (w1, b1), (w2, b2) = loc
    (g1, c1), (g2, c2) = glob
    b1, b2, c1, c2 = (v.reshape(1, -1) for v in (b1, b2, c1, c2))
    C2g = g2.shape[1]
    kfn = functools.partial(_conv_kernel, r2=r2, kn=kn)
    full = lambda a: pl.BlockSpec(a.shape, lambda i: (0,) * a.ndim)
    return pl.pallas_call(
        kfn,
        grid=(nb,),
        in_specs=[pl.BlockSpec((128 * kn, xin.shape[1]), lambda i: (i, 0)),
                  pl.BlockSpec((128, kn), lambda i: (i, 0)),
                  full(w1), full(b1), full(w2), full(b2),
                  full(g1), full(c1), full(g2), full(c2)],
        out_specs=pl.BlockSpec((128, C2g), lambda i: (i, 0)),
        out_shape=jax.ShapeDtypeStruct((NQ, C2g), jnp.float32),
    )(xin, dsel, w1, b1, w2, b2, g1, c1, g2, c2)


def _fp_kernel(*refs, has_dec):
    if has_dec:
        (xg_ref, ds_ref, zs_ref, wa_ref, wb_ref, b1_ref,
         w2_ref, b2_ref, wd_ref, bd_ref, o_ref) = refs
    else:
        (xg_ref, ds_ref, zs_ref, wa_ref, wb_ref, b1_ref,
         w2_ref, b2_ref, o_ref) = refs
        wd_ref = bd_ref = None
    w = 1.0 / jnp.maximum(ds_ref[...], 1e-16)
    num = (xg_ref[:, 0, :] * w[:, 0:1]
           + xg_ref[:, 1, :] * w[:, 1:2]
           + xg_ref[:, 2, :] * w[:, 2:3])
    zi = num / (w[:, 0:1] + w[:, 1:2] + w[:, 2:3])
    h = _relu(_dot(zi, wa_ref[...]) + _dot(zs_ref[...], wb_ref[...])
              + b1_ref[...])
    h = _relu(_dot(h, w2_ref[...]) + b2_ref[...])
    if wd_ref is not None:
        o_ref[...] = _dot(h, wd_ref[...]) + bd_ref[...]
    else:
        o_ref[...] = h


def _fp_call(xg, dsel, zskip, fp, decoder=None, block=128):
    NQ, _, C = xg.shape
    nb = NQ // block
    (w1, b1), (w2, b2) = fp
    b1 = b1.reshape(1, -1)
    b2 = b2.reshape(1, -1)
    wa = w1[:C]
    wb = w1[C:]
    full = lambda a: pl.BlockSpec(a.shape, lambda i: (0,) * a.ndim)
    ins = [xg, dsel, zskip, wa, wb, b1, w2, b2]
    in_specs = [pl.BlockSpec((block, 3, C), lambda i: (i, 0, 0)),
                pl.BlockSpec((block, 3), lambda i: (i, 0)),
                pl.BlockSpec((block, zskip.shape[1]), lambda i: (i, 0)),
                full(wa), full(wb), full(b1), full(w2), full(b2)]
    if decoder is not None:
        wd, bd = decoder
        ins += [wd, bd]
        in_specs += [full(wd), full(bd)]
        cout = wd.shape[1]
        kfn = functools.partial(_fp_kernel, has_dec=True)
    else:
        cout = w2.shape[1]
        kfn = functools.partial(_fp_kernel, has_dec=False)
    return pl.pallas_call(
        kfn,
        grid=(nb,),
        in_specs=in_specs,
        out_specs=pl.BlockSpec((block, cout), lambda i: (i, 0)),
        out_shape=jax.ShapeDtypeStruct((NQ, cout), jnp.float32),
    )(*ins)


def _enc_kernel(x_ref, w_ref, b_ref, o_ref):
    o_ref[...] = _relu(_dot(x_ref[...], w_ref[...]) + b_ref[...])


def _enc_call(xp, w, b, block=1024):
    Np, Cin = xp.shape
    nb = Np // block
    full = lambda a: pl.BlockSpec(a.shape, lambda i: (0,) * a.ndim)
    return pl.pallas_call(
        _enc_kernel,
        grid=(nb,),
        in_specs=[pl.BlockSpec((block, Cin), lambda i: (i, 0)),
                  full(w), full(b)],
        out_specs=pl.BlockSpec((block, w.shape[1]), lambda i: (i, 0)),
        out_shape=jax.ShapeDtypeStruct((Np, w.shape[1]), jnp.float32),
    )(xp, w, b)


def _pad_to(n, m):
    return ((n + m - 1) // m) * m


def _pad_flat(v, np_, fill):
    n = v.shape[0]
    return jnp.concatenate([v, jnp.full((np_ - n,), fill, v.dtype)])


def kernel(x, params):
    N0 = x.shape[0]
    n1 = int(math.ceil(0.25 * N0))
    n2 = int(math.ceil(0.25 * n1))
    N0p = _pad_to(N0, 1024)
    n1p = _pad_to(n1, 128)
    n2p = _pad_to(n2, 128)

    pos0 = x[:, :2]
    p0x = _pad_flat(pos0[:, 0], N0p, _PAD)
    p0y = _pad_flat(pos0[:, 1], N0p, _PAD)

    xp = jnp.concatenate(
        [x, jnp.zeros((N0p - N0, x.shape[1]), x.dtype)], axis=0)
    (we, be) = params['encoder'][0]
    z0 = _enc_call(xp, we, be.reshape(1, -1))

    s1x, s1y = _fps_call(p0x.reshape(-1, 128), p0y.reshape(-1, 128),
                         N0, n1, n1p // 128)
    valid1 = jnp.arange(n1p) < n1
    p1x = jnp.where(valid1, s1x.reshape(-1), _PAD)
    p1y = jnp.where(valid1, s1y.reshape(-1), _PAD)

    nb1, ds1 = _topk_call(p1x.reshape(-1, 1), p1y.reshape(-1, 1),
                          p0x.reshape(1, -1), p0y.reshape(1, -1), 64)
    xj = z0[nb1]
    relx = p0x[nb1] - p1x[:, None]
    rely = p0y[nb1] - p1y[:, None]
    xin = jnp.concatenate([xj, relx[..., None], rely[..., None]], axis=-1)
    xin = xin.reshape(n1p * 64, -1)
    z1 = _conv_call(xin, ds1, params['local0'],
                    params['global0'], 0.1 * 0.1)

    s2x, s2y = _fps_call(p1x.reshape(-1, 128), p1y.reshape(-1, 128),
                         n1, n2, n2p // 128)
    valid2 = jnp.arange(n2p) < n2
    p2x = jnp.where(valid2, s2x.reshape(-1), _PAD)
    p2y = jnp.where(valid2, s2y.reshape(-1), _PAD)

    nb2, ds2 = _topk_call(p2x.reshape(-1, 1), p2y.reshape(-1, 1),
                          p1x.reshape(1, -1), p1y.reshape(1, -1), 64)
    xj2 = z1[nb2]
    relx2 = p1x[nb2] - p2x[:, None]
    rely2 = p1y[nb2] - p2y[:, None]
    xin2 = jnp.concatenate([xj2, relx2[..., None], rely2[..., None]],
                           axis=-1).reshape(n2p * 64, -1)
    z2 = _conv_call(xin2, ds2, params['local1'],
                    params['global1'], 0.2 * 0.2)

    if1, df1 = _topk_call(p1x.reshape(-1, 1), p1y.reshape(-1, 1),
                          p2x.reshape(1, -1), p2y.reshape(1, -1), 3)
    xg1 = z2[if1]
    zf1 = _fp_call(xg1, df1, z1, params['fp1'])

    if0, df0 = _topk_call(p0x.reshape(-1, 1), p0y.reshape(-1, 1),
                          p1x.reshape(1, -1), p1y.reshape(1, -1), 3)
    xg0 = zf1[if0]
    wd, bd = params['decoder']
    out = _fp_call(xg0, df0, z0, params['fp0'],
                   decoder=(wd, bd.reshape(1, -1)))
    return out[:N0]


# speedup vs baseline: 4.1045x; 1.1081x over previous
import functools
import math

import jax
import jax.numpy as jnp
from jax import lax
from jax.experimental import pallas as pl
from jax.experimental.pallas import tpu as pltpu

_BIGF = 3.0e38
_NEG = -1.0e30
_PAD = 1.0e9


def _relu(v):
    return jnp.maximum(v, 0.0)


def _dot(a, b):
    return jnp.dot(a, b, preferred_element_type=jnp.float32)


def _fps_kernel(px_ref, py_ref, ox_ref, oy_ref, *, n_valid, num_sel):
    R = px_ref.shape[0]
    OR = ox_ref.shape[0]
    iota = (lax.broadcasted_iota(jnp.int32, (R, 128), 0) * 128
            + lax.broadcasted_iota(jnp.int32, (R, 128), 1))
    oiota = (lax.broadcasted_iota(jnp.int32, (OR, 128), 0) * 128
             + lax.broadcasted_iota(jnp.int32, (OR, 128), 1))
    px = px_ref[...]
    py = py_ref[...]
    x0 = px[0, 0]
    y0 = py[0, 0]
    d0 = (px - x0) ** 2 + (py - y0) ** 2
    dists = jnp.where(iota < n_valid, d0, -1.0)
    selx = jnp.where(oiota == 0, x0, 0.0)
    sely = jnp.where(oiota == 0, y0, 0.0)

    def body(i, carry):
        dists, selx, sely = carry
        m = jnp.max(dists)
        nxt = jnp.min(jnp.where(dists == m, iota, jnp.int32(2 ** 30)))
        gx = jnp.sum(jnp.where(iota == nxt, px, 0.0))
        gy = jnp.sum(jnp.where(iota == nxt, py, 0.0))
        d = (px - gx) ** 2 + (py - gy) ** 2
        dists = jnp.minimum(dists, d)
        selx = jnp.where(oiota == i, gx, selx)
        sely = jnp.where(oiota == i, gy, sely)
        return dists, selx, sely

    dists, selx, sely = lax.fori_loop(1, num_sel, body, (dists, selx, sely))
    ox_ref[...] = selx
    oy_ref[...] = sely


def _fps_call(px, py, n_valid, num_sel, out_rows):
    kfn = functools.partial(_fps_kernel, n_valid=n_valid, num_sel=num_sel)
    out_sds = jax.ShapeDtypeStruct((out_rows, 128), jnp.float32)
    return pl.pallas_call(
        kfn,
        out_shape=(out_sds, out_sds),
    )(px, py)


def _topk_kernel(qx_ref, qy_ref, sx_ref, sy_ref, idx_ref, dsel_ref, d2_ref,
                 *, k):
    S = sx_ref.shape[1]
    qx = qx_ref[...]
    qy = qy_ref[...]
    sx = sx_ref[...]
    sy = sy_ref[...]
    d2_ref[...] = (qx - sx) ** 2 + (qy - sy) ** 2
    siota = lax.broadcasted_iota(jnp.int32, (1, S), 1)
    for r in range(k):
        d2 = d2_ref[...]
        best = jnp.min(d2, axis=1, keepdims=True)
        bidx = jnp.min(jnp.where(d2 == best, siota, jnp.int32(2 ** 30)),
                       axis=1, keepdims=True)
        idx_ref[:, r:r + 1] = bidx
        dsel_ref[:, r:r + 1] = best
        d2_ref[...] = jnp.where(siota == bidx, _BIGF, d2)


def _topk64_kernel(qx_ref, qy_ref, sx_ref, sy_ref, idx_ref, dsel_ref,
                   key_ref, *, k):
    S = sx_ref.shape[1]
    qx = qx_ref[...]
    qy = qy_ref[...]
    d2 = (qx - sx_ref[...]) ** 2 + (qy - sy_ref[...]) ** 2
    siota = lax.broadcasted_iota(jnp.int32, (1, S), 1)
    hi = jnp.int32(-16384)
    key_ref[...] = (lax.bitcast_convert_type(d2, jnp.int32) & hi) | siota
    for r in range(k):
        keys = key_ref[...]
        bkey = jnp.min(keys, axis=1, keepdims=True)
        idx_ref[:, r:r + 1] = bkey & jnp.int32(16383)
        dsel_ref[:, r:r + 1] = lax.bitcast_convert_type(
            bkey & hi, jnp.float32)
        key_ref[...] = jnp.where(keys == bkey, jnp.int32(2 ** 31 - 1), keys)


def _topk64_call(qx, qy, sx, sy, k):
    NQ = qx.shape[0]
    S = sx.shape[1]
    nb = NQ // 128
    kfn = functools.partial(_topk64_kernel, k=k)
    qspec = pl.BlockSpec((128, 1), lambda i: (i, 0))
    sspec = pl.BlockSpec((1, S), lambda i: (0, 0))
    return pl.pallas_call(
        kfn,
        grid=(nb,),
        in_specs=[qspec, qspec, sspec, sspec],
        out_specs=(pl.BlockSpec((128, k), lambda i: (i, 0)),
                   pl.BlockSpec((128, k), lambda i: (i, 0))),
        out_shape=(jax.ShapeDtypeStruct((NQ, k), jnp.int32),
                   jax.ShapeDtypeStruct((NQ, k), jnp.float32)),
        scratch_shapes=[pltpu.VMEM((128, S), jnp.int32)],
    )(qx, qy, sx, sy)


def _topk_call(qx, qy, sx, sy, k):
    NQ = qx.shape[0]
    S = sx.shape[1]
    nb = NQ // 128
    kfn = functools.partial(_topk_kernel, k=k)
    qspec = pl.BlockSpec((128, 1), lambda i: (i, 0))
    sspec = pl.BlockSpec((1, S), lambda i: (0, 0))
    return pl.pallas_call(
        kfn,
        grid=(nb,),
        in_specs=[qspec, qspec, sspec, sspec],
        out_specs=(pl.BlockSpec((128, k), lambda i: (i, 0)),
                   pl.BlockSpec((128, k), lambda i: (i, 0))),
        out_shape=(jax.ShapeDtypeStruct((NQ, k), jnp.int32),
                   jax.ShapeDtypeStruct((NQ, k), jnp.float32)),
        scratch_shapes=[pltpu.VMEM((128, S), jnp.float32)],
    )(qx, qy, sx, sy)


def _conv_kernel(xin_ref, dsel_ref, w1_ref, b1_ref, w2_ref, b2_ref,
                 g1_ref, c1_ref, g2_ref, c2_ref, o_ref, *, r2, kn):
    h = _relu(_dot(xin_ref[...], w1_ref[...]) + b1_ref[...])
    h = _relu(_dot(h, w2_ref[...]) + b2_ref[...])
    C2 = h.shape[1]
    h3 = h.reshape(128, kn, C2)
    mask = dsel_ref[...] <= r2
    hm = jnp.full((128, C2), _NEG, jnp.float32)
    for j in range(kn):
        hm = jnp.maximum(hm, jnp.where(mask[:, j:j + 1], h3[:, j, :], _NEG))
    g = _relu(_dot(hm, g1_ref[...]) + c1_ref[...])
    o_ref[...] = _relu(_dot(g, g2_ref[...]) + c2_ref[...])


def _conv_call(xin, dsel, loc, glob, r2):
    NQ, kn = dsel.shape
    nb = NQ // 128
    (w1, b1), (w2, b2) = loc
    (g1, c1), (g2, c2) = glob
    b1, b2, c1, c2 = (v.reshape(1, -1) for v in (b1, b2, c1, c2))
    C2g = g2.shape[1]
    kfn = functools.partial(_conv_kernel, r2=r2, kn=kn)
    full = lambda a: pl.BlockSpec(a.shape, lambda i: (0,) * a.ndim)
    return pl.pallas_call(
        kfn,
        grid=(nb,),
        in_specs=[pl.BlockSpec((128 * kn, xin.shape[1]), lambda i: (i, 0)),
                  pl.BlockSpec((128, kn), lambda i: (i, 0)),
                  full(w1), full(b1), full(w2), full(b2),
                  full(g1), full(c1), full(g2), full(c2)],
        out_specs=pl.BlockSpec((128, C2g), lambda i: (i, 0)),
        out_shape=jax.ShapeDtypeStruct((NQ, C2g), jnp.float32),
    )(xin, dsel, w1, b1, w2, b2, g1, c1, g2, c2)


def _fp_kernel(*refs, has_dec):
    if has_dec:
        (xg_ref, ds_ref, zs_ref, wa_ref, wb_ref, b1_ref,
         w2_ref, b2_ref, wd_ref, bd_ref, o_ref) = refs
    else:
        (xg_ref, ds_ref, zs_ref, wa_ref, wb_ref, b1_ref,
         w2_ref, b2_ref, o_ref) = refs
        wd_ref = bd_ref = None
    w = 1.0 / jnp.maximum(ds_ref[...], 1e-16)
    num = (xg_ref[:, 0, :] * w[:, 0:1]
           + xg_ref[:, 1, :] * w[:, 1:2]
           + xg_ref[:, 2, :] * w[:, 2:3])
    zi = num / (w[:, 0:1] + w[:, 1:2] + w[:, 2:3])
    h = _relu(_dot(zi, wa_ref[...]) + _dot(zs_ref[...], wb_ref[...])
              + b1_ref[...])
    h = _relu(_dot(h, w2_ref[...]) + b2_ref[...])
    if wd_ref is not None:
        o_ref[...] = _dot(h, wd_ref[...]) + bd_ref[...]
    else:
        o_ref[...] = h


def _fp_call(xg, dsel, zskip, fp, decoder=None, block=128):
    NQ, _, C = xg.shape
    nb = NQ // block
    (w1, b1), (w2, b2) = fp
    b1 = b1.reshape(1, -1)
    b2 = b2.reshape(1, -1)
    wa = w1[:C]
    wb = w1[C:]
    full = lambda a: pl.BlockSpec(a.shape, lambda i: (0,) * a.ndim)
    ins = [xg, dsel, zskip, wa, wb, b1, w2, b2]
    in_specs = [pl.BlockSpec((block, 3, C), lambda i: (i, 0, 0)),
                pl.BlockSpec((block, 3), lambda i: (i, 0)),
                pl.BlockSpec((block, zskip.shape[1]), lambda i: (i, 0)),
                full(wa), full(wb), full(b1), full(w2), full(b2)]
    if decoder is not None:
        wd, bd = decoder
        ins += [wd, bd]
        in_specs += [full(wd), full(bd)]
        cout = wd.shape[1]
        kfn = functools.partial(_fp_kernel, has_dec=True)
    else:
        cout = w2.shape[1]
        kfn = functools.partial(_fp_kernel, has_dec=False)
    return pl.pallas_call(
        kfn,
        grid=(nb,),
        in_specs=in_specs,
        out_specs=pl.BlockSpec((block, cout), lambda i: (i, 0)),
        out_shape=jax.ShapeDtypeStruct((NQ, cout), jnp.float32),
    )(*ins)


def _enc_kernel(x_ref, w_ref, b_ref, o_ref):
    o_ref[...] = _relu(_dot(x_ref[...], w_ref[...]) + b_ref[...])


def _enc_call(xp, w, b, block=1024):
    Np, Cin = xp.shape
    nb = Np // block
    full = lambda a: pl.BlockSpec(a.shape, lambda i: (0,) * a.ndim)
    return pl.pallas_call(
        _enc_kernel,
        grid=(nb,),
        in_specs=[pl.BlockSpec((block, Cin), lambda i: (i, 0)),
                  full(w), full(b)],
        out_specs=pl.BlockSpec((block, w.shape[1]), lambda i: (i, 0)),
        out_shape=jax.ShapeDtypeStruct((Np, w.shape[1]), jnp.float32),
    )(xp, w, b)


def _pad_to(n, m):
    return ((n + m - 1) // m) * m


def _pad_flat(v, np_, fill):
    n = v.shape[0]
    return jnp.concatenate([v, jnp.full((np_ - n,), fill, v.dtype)])


def kernel(x, params):
    N0 = x.shape[0]
    n1 = int(math.ceil(0.25 * N0))
    n2 = int(math.ceil(0.25 * n1))
    N0p = _pad_to(N0, 1024)
    n1p = _pad_to(n1, 128)
    n2p = _pad_to(n2, 128)

    pos0 = x[:, :2]
    p0x = _pad_flat(pos0[:, 0], N0p, _PAD)
    p0y = _pad_flat(pos0[:, 1], N0p, _PAD)

    xp = jnp.concatenate(
        [x, jnp.zeros((N0p - N0, x.shape[1]), x.dtype)], axis=0)
    (we, be) = params['encoder'][0]
    z0 = _enc_call(xp, we, be.reshape(1, -1))

    s1x, s1y = _fps_call(p0x.reshape(-1, 128), p0y.reshape(-1, 128),
                         N0, n1, n1p // 128)
    valid1 = jnp.arange(n1p) < n1
    p1x = jnp.where(valid1, s1x.reshape(-1), _PAD)
    p1y = jnp.where(valid1, s1y.reshape(-1), _PAD)

    nb1, ds1 = _topk64_call(p1x.reshape(-1, 1), p1y.reshape(-1, 1),
                            p0x.reshape(1, -1), p0y.reshape(1, -1), 64)
    xj = z0[nb1]
    relx = p0x[nb1] - p1x[:, None]
    rely = p0y[nb1] - p1y[:, None]
    xin = jnp.concatenate([xj, relx[..., None], rely[..., None]], axis=-1)
    xin = xin.reshape(n1p * 64, -1)
    z1 = _conv_call(xin, ds1, params['local0'],
                    params['global0'], 0.1 * 0.1)

    s2x, s2y = _fps_call(p1x.reshape(-1, 128), p1y.reshape(-1, 128),
                         n1, n2, n2p // 128)
    valid2 = jnp.arange(n2p) < n2
    p2x = jnp.where(valid2, s2x.reshape(-1), _PAD)
    p2y = jnp.where(valid2, s2y.reshape(-1), _PAD)

    nb2, ds2 = _topk64_call(p2x.reshape(-1, 1), p2y.reshape(-1, 1),
                            p1x.reshape(1, -1), p1y.reshape(1, -1), 64)
    xj2 = z1[nb2]
    relx2 = p1x[nb2] - p2x[:, None]
    rely2 = p1y[nb2] - p2y[:, None]
    xin2 = jnp.concatenate([xj2, relx2[..., None], rely2[..., None]],
                           axis=-1).reshape(n2p * 64, -1)
    z2 = _conv_call(xin2, ds2, params['local1'],
                    params['global1'], 0.2 * 0.2)

    if1, df1 = _topk_call(p1x.reshape(-1, 1), p1y.reshape(-1, 1),
                          p2x.reshape(1, -1), p2y.reshape(1, -1), 3)
    xg1 = z2[if1]
    zf1 = _fp_call(xg1, df1, z1, params['fp1'])

    if0, df0 = _topk_call(p0x.reshape(-1, 1), p0y.reshape(-1, 1),
                          p1x.reshape(1, -1), p1y.reshape(1, -1), 3)
    xg0 = zf1[if0]
    wd, bd = params['decoder']
    out = _fp_call(xg0, df0, z0, params['fp0'],
                   decoder=(wd, bd.reshape(1, -1)))
    return out[:N0]
